# baseline (speedup 1.0000x reference)
import functools

import jax
import jax.numpy as jnp
from jax import lax
from jax.experimental import pallas as pl
from jax.experimental.pallas import tpu as pltpu
from jax.experimental.pallas import tpu_sc as plsc

_N = 10000
_G = 100
_n = 100
_DF = 128
_DH = 128
_DL = 64
_E = 160000
_W = 128
_TBL = _N * _W
_TBL_PAD = _TBL + 256
_NW = 32
_EPW = _E // _NW
_CHUNKS = 313
_ROWS = 40
_ZCH = 32000
_NCH = _TBL // _ZCH
_KMAX = 3
_GB = 10


def _sc_counts_body(ei_hbm, out0_hbm, out1_hbm, src_v, dst_v,
                    idx_v, val_v, zbuf, table_sh, sem_s, sem_sc):
    cid = lax.axis_index("c")
    sid = lax.axis_index("s")
    wid = cid * 16 + sid
    base = wid * _EPW

    ones16 = jnp.ones((16,), jnp.float32)
    zeros16 = jnp.zeros((16,), jnp.float32)
    trash16 = jnp.full((16,), _TBL, jnp.int32)
    lane = lax.broadcasted_iota(jnp.int32, (16,), 0)

    cp_s = pltpu.async_copy(ei_hbm.at[pl.ds(base, _EPW)],
                            src_v.at[pl.ds(0, _EPW)], sem_s)
    cp_d = pltpu.async_copy(ei_hbm.at[pl.ds(_E + base, _EPW)],
                            dst_v.at[pl.ds(0, _EPW)], sem_s)

    with jax.named_scope("p1_zbuf"):
        def _zb(i, carry):
            for c in range(8):
                zbuf[pl.ds(i * 128 + c * 16, 16)] = zeros16
            return carry
        lax.fori_loop(0, _ZCH // 128, _zb, 0)

    with jax.named_scope("p2_zerotable"):
        for k in range(_KMAX):
            ch = sid + 16 * k

            @pl.when(ch < _NCH)
            def _():
                pltpu.sync_copy(zbuf, table_sh.at[pl.ds(ch * _ZCH, _ZCH)])

        @pl.when(sid == 0)
        def _():
            pltpu.sync_copy(zbuf.at[pl.ds(0, 256)],
                            table_sh.at[pl.ds(_TBL, 256)])

    for i in range(8):
        val_v[pl.ds(i * 16, 16)] = ones16

    with jax.named_scope("p3_edgewait"):
        cp_s.wait()
        cp_d.wait()

    def _ix(i, carry):
        row = i // 8
        col = (i % 8) * 16
        s16 = src_v[pl.ds(i * 16, 16)]
        d16 = dst_v[pl.ds(i * 16, 16)]
        fl = d16 * _W + lax.rem(s16, _n)
        pos = i * 16 + lane
        fl = jnp.where(pos < _EPW, fl, _TBL)
        idx_v[row, pl.ds(col, 16)] = fl
        return carry
    with jax.named_scope("p4_idx"):
        lax.fori_loop(0, _CHUNKS, _ix, 0)
        for c in range(1, 8):
            idx_v[_ROWS - 1, pl.ds(c * 16, 16)] = trash16

    with jax.named_scope("p5_bar1"):
        plsc.subcore_barrier()

    with jax.named_scope("p6_scatter"):
        scps = [pltpu.async_copy(val_v, table_sh.at[idx_v.at[j]], sem_sc,
                                 add=True)
                for j in range(_ROWS)]
        for cp in scps:
            cp.wait()

    with jax.named_scope("p7_bar2"):
        plsc.subcore_barrier()

    with jax.named_scope("p8_copyout"):
        for k in range(_KMAX):
            ch = sid + 16 * k

            @pl.when(ch < _NCH)
            def _():
                pltpu.sync_copy(table_sh.at[pl.ds(ch * _ZCH, _ZCH)], zbuf)

                @pl.when(cid == 0)
                def _():
                    pltpu.sync_copy(zbuf,
                                    out0_hbm.at[pl.ds(ch * _ZCH, _ZCH)])

                @pl.when(cid == 1)
                def _():
                    pltpu.sync_copy(zbuf,
                                    out1_hbm.at[pl.ds(ch * _ZCH, _ZCH)])


@jax.jit
def _sc_counts(ei_flat):
    mesh = plsc.VectorSubcoreMesh(core_axis_name="c", subcore_axis_name="s")
    f = functools.partial(
        pl.kernel,
        mesh=mesh,
        out_type=(jax.ShapeDtypeStruct((_TBL,), jnp.float32),
                  jax.ShapeDtypeStruct((_TBL,), jnp.float32)),
        scratch_types=[
            pltpu.VMEM((_EPW + 16,), jnp.int32),
            pltpu.VMEM((_EPW + 16,), jnp.int32),
            pltpu.VMEM((_ROWS, 128), jnp.int32),
            pltpu.VMEM((128,), jnp.float32),
            pltpu.VMEM((_ZCH,), jnp.float32),
            pltpu.VMEM_SHARED((_TBL_PAD,), jnp.float32),
            pltpu.SemaphoreType.DMA,
            pltpu.SemaphoreType.DMA,
        ],
    )(_sc_counts_body)
    return f(ei_flat)


def _xw1_body(x_ref, w1_ref, y_ref):
    y_ref[...] = jnp.dot(x_ref[...], w1_ref[...],
                         preferred_element_type=jnp.float32)


@jax.jit
def _xw1(x, W1):
    blk = _GB * _n
    return pl.pallas_call(
        _xw1_body,
        grid=(_G // _GB,),
        in_specs=[
            pl.BlockSpec((blk, _DF), lambda g: (g, 0)),
            pl.BlockSpec((_DF, _DH), lambda g: (0, 0)),
        ],
        out_specs=pl.BlockSpec((blk, _DH), lambda g: (g, 0)),
        out_shape=jax.ShapeDtypeStruct((_N, _DH), jnp.float32),
    )(x, W1)


def _tc_elbo_body(cnt0_ref, cnt1_ref, y_ref, eps_ref, b1_ref,
                  wmu_ref, bmu_ref, wlv_ref, blv_ref, out_ref):
    g = pl.program_id(0)
    ones_col = jnp.ones((_n, 1), jnp.float32)
    mm = (((1,), (0,)), ((), ()))
    dl = (((1,), (1,)), ((), ()))
    ri = lax.broadcasted_iota(jnp.int32, (_n, _n), 0)
    ci = lax.broadcasted_iota(jnp.int32, (_n, _n), 1)
    lower = ri > ci

    cts = [(cnt0_ref[pl.ds(b * _n, _n), :]
            + cnt1_ref[pl.ds(b * _n, _n), :])[:, 0:_n] for b in range(_GB)]
    ygs = [y_ref[pl.ds(b * _n, _n), :] for b in range(_GB)]
    aggs = [lax.dot_general(cts[b], ygs[b], mm,
                            preferred_element_type=jnp.float32)
            for b in range(_GB)]
    degs = [lax.dot_general(cts[b], ones_col, mm,
                            preferred_element_type=jnp.float32) + 1.0
            for b in range(_GB)]
    hs = [jnp.maximum(
        (ygs[b] + aggs[b]) / degs[b] + b1_ref[...], 0.0)
        for b in range(_GB)]
    mus = [jnp.dot(hs[b], wmu_ref[...], preferred_element_type=jnp.float32)
           + bmu_ref[...] for b in range(_GB)]
    lvs = [jnp.dot(hs[b], wlv_ref[...], preferred_element_type=jnp.float32)
           + blv_ref[...] for b in range(_GB)]
    sigs = [jnp.exp(0.5 * lvs[b]) for b in range(_GB)]
    zs = [mus[b] + sigs[b] * eps_ref[pl.ds(b * _n, _n), :]
          for b in range(_GB)]
    Ls = [lax.dot_general(zs[b], zs[b], dl,
                          preferred_element_type=jnp.float32)
          for b in range(_GB)]

    kl_acc = jnp.zeros((_n, _DL), jnp.float32)
    val_acc = jnp.zeros((_n, _n), jnp.float32)
    for b in range(_GB):
        kl_acc += mus[b] * mus[b] + sigs[b] * sigs[b] - lvs[b]
        L = Ls[b]
        sp = jnp.maximum(L, 0.0) + jnp.log1p(jnp.exp(-jnp.abs(L)))
        val_acc += jnp.where(cts[b] > 0.0, L, 0.0) - sp

    klsum = 0.5 * (jnp.sum(kl_acc) - _GB * _n * _DL)
    logp = jnp.sum(jnp.where(lower, val_acc, 0.0))
    total = klsum - logp

    @pl.when(g == 0)
    def _():
        out_ref[0, 0] = 0.0

    out_ref[0, 0] += total * (1.0 / _G)


@jax.jit
def _tc_elbo(cnt0, cnt1, y, eps, b1, Wmu, bmu, Wlv, blv):
    blk = _GB * _n
    return pl.pallas_call(
        _tc_elbo_body,
        grid=(_G // _GB,),
        in_specs=[
            pl.BlockSpec((blk, _W), lambda g: (g, 0)),
            pl.BlockSpec((blk, _W), lambda g: (g, 0)),
            pl.BlockSpec((blk, _DH), lambda g: (g, 0)),
            pl.BlockSpec((blk, _DL), lambda g: (g, 0)),
            pl.BlockSpec((1, _DH), lambda g: (0, 0)),
            pl.BlockSpec((_DH, _DL), lambda g: (0, 0)),
            pl.BlockSpec((1, _DL), lambda g: (0, 0)),
            pl.BlockSpec((_DH, _DL), lambda g: (0, 0)),
            pl.BlockSpec((1, _DL), lambda g: (0, 0)),
        ],
        out_specs=pl.BlockSpec(memory_space=pltpu.SMEM),
        out_shape=jax.ShapeDtypeStruct((1, 1), jnp.float32),
    )(cnt0, cnt1, y, eps, b1, Wmu, bmu, Wlv, blv)


def kernel(x, edge_index, batch, eps, W1, b1, Wmu, bmu, Wlv, blv, pairs):
    c0, c1 = _sc_counts(edge_index.reshape(2 * _E))
    y = _xw1(x, W1)
    out = _tc_elbo(c0.reshape(_N, _W), c1.reshape(_N, _W), y, eps,
                   b1.reshape(1, _DH), Wmu, bmu.reshape(1, _DL),
                   Wlv, blv.reshape(1, _DL))
    return out[0, 0]


# speedup vs baseline: 1.0051x; 1.0051x over previous
import functools

import jax
import jax.numpy as jnp
from jax import lax
from jax.experimental import pallas as pl
from jax.experimental.pallas import tpu as pltpu
from jax.experimental.pallas import tpu_sc as plsc

_N = 10000
_G = 100
_n = 100
_DF = 128
_DH = 128
_DL = 64
_E = 160000
_W = 128
_TBL = _N * _W
_TBL_PAD = _TBL + 256
_NW = 32
_EPW = _E // _NW
_CHUNKS = 313
_ROWS = 40
_ZCH = 32000
_NCH = _TBL // _ZCH
_KMAX = 3
_OCH = 16000
_OK = 5
_GB = 20


def _sc_counts_body(ei_hbm, out0_hbm, out1_hbm, src_v, dst_v,
                    idx_v, val_v, zbuf, table_sh, sem_s, sem_sc):
    cid = lax.axis_index("c")
    sid = lax.axis_index("s")
    wid = cid * 16 + sid
    base = wid * _EPW

    ones16 = jnp.ones((16,), jnp.float32)
    zeros16 = jnp.zeros((16,), jnp.float32)
    trash16 = jnp.full((16,), _TBL, jnp.int32)
    lane = lax.broadcasted_iota(jnp.int32, (16,), 0)

    cp_s = pltpu.async_copy(ei_hbm.at[pl.ds(base, _EPW)],
                            src_v.at[pl.ds(0, _EPW)], sem_s)
    cp_d = pltpu.async_copy(ei_hbm.at[pl.ds(_E + base, _EPW)],
                            dst_v.at[pl.ds(0, _EPW)], sem_s)

    with jax.named_scope("p1_zbuf"):
        def _zb(i, carry):
            for c in range(8):
                zbuf[pl.ds(i * 128 + c * 16, 16)] = zeros16
            return carry
        lax.fori_loop(0, _ZCH // 128, _zb, 0)

    with jax.named_scope("p2_zerotable"):
        for k in range(_KMAX):
            ch = sid + 16 * k

            @pl.when(ch < _NCH)
            def _():
                pltpu.sync_copy(zbuf, table_sh.at[pl.ds(ch * _ZCH, _ZCH)])

        @pl.when(sid == 0)
        def _():
            pltpu.sync_copy(zbuf.at[pl.ds(0, 256)],
                            table_sh.at[pl.ds(_TBL, 256)])

    for i in range(8):
        val_v[pl.ds(i * 16, 16)] = ones16

    with jax.named_scope("p3_edgewait"):
        cp_s.wait()
        cp_d.wait()

    def _ixrow(r, carry):
        base16 = r * 128
        for c in range(8):
            off = base16 + c * 16
            s16 = src_v[pl.ds(off, 16)]
            d16 = dst_v[pl.ds(off, 16)]
            idx_v[r, pl.ds(c * 16, 16)] = d16 * _W + lax.rem(s16, _n)
        return carry
    with jax.named_scope("p4_idx"):
        lax.fori_loop(0, _ROWS - 1, _ixrow, 0)
        s16 = src_v[pl.ds((_ROWS - 1) * 128, 16)]
        d16 = dst_v[pl.ds((_ROWS - 1) * 128, 16)]
        fl = d16 * _W + lax.rem(s16, _n)
        fl = jnp.where(lane < _EPW - (_ROWS - 1) * 128, fl, _TBL)
        idx_v[_ROWS - 1, pl.ds(0, 16)] = fl
        for c in range(1, 8):
            idx_v[_ROWS - 1, pl.ds(c * 16, 16)] = trash16

    with jax.named_scope("p5_bar1"):
        plsc.subcore_barrier()

    with jax.named_scope("p6_scatter"):
        scps = [pltpu.async_copy(val_v, table_sh.at[idx_v.at[j]], sem_sc,
                                 add=True)
                for j in range(_ROWS)]
        for cp in scps:
            cp.wait()

    with jax.named_scope("p7_bar2"):
        plsc.subcore_barrier()

    with jax.named_scope("p8_copyout"):
        for k in range(_KMAX):
            ch = sid + 16 * k

            @pl.when(ch < _NCH)
            def _():
                pltpu.sync_copy(table_sh.at[pl.ds(ch * _ZCH, _ZCH)], zbuf)

                @pl.when(cid == 0)
                def _():
                    pltpu.sync_copy(zbuf,
                                    out0_hbm.at[pl.ds(ch * _ZCH, _ZCH)])

                @pl.when(cid == 1)
                def _():
                    pltpu.sync_copy(zbuf,
                                    out1_hbm.at[pl.ds(ch * _ZCH, _ZCH)])


@jax.jit
def _sc_counts(ei_flat):
    mesh = plsc.VectorSubcoreMesh(core_axis_name="c", subcore_axis_name="s")
    f = functools.partial(
        pl.kernel,
        mesh=mesh,
        out_type=(jax.ShapeDtypeStruct((_TBL,), jnp.float32),
                  jax.ShapeDtypeStruct((_TBL,), jnp.float32)),
        scratch_types=[
            pltpu.VMEM((_EPW + 16,), jnp.int32),
            pltpu.VMEM((_EPW + 16,), jnp.int32),
            pltpu.VMEM((_ROWS, 128), jnp.int32),
            pltpu.VMEM((128,), jnp.float32),
            pltpu.VMEM((_ZCH,), jnp.float32),
            pltpu.VMEM_SHARED((_TBL_PAD,), jnp.float32),
            pltpu.SemaphoreType.DMA,
            pltpu.SemaphoreType.DMA,
        ],
    )(_sc_counts_body)
    return f(ei_flat)


def _xw1_body(x_ref, w1_ref, y_ref):
    y_ref[...] = jnp.dot(x_ref[...], w1_ref[...],
                         preferred_element_type=jnp.float32)


@jax.jit
def _xw1(x, W1):
    blk = _GB * _n
    return pl.pallas_call(
        _xw1_body,
        grid=(_G // _GB,),
        in_specs=[
            pl.BlockSpec((blk, _DF), lambda g: (g, 0)),
            pl.BlockSpec((_DF, _DH), lambda g: (0, 0)),
        ],
        out_specs=pl.BlockSpec((blk, _DH), lambda g: (g, 0)),
        out_shape=jax.ShapeDtypeStruct((_N, _DH), jnp.float32),
    )(x, W1)


def _tc_elbo_body(cnt0_ref, cnt1_ref, y_ref, eps_ref, b1_ref,
                  wmu_ref, bmu_ref, wlv_ref, blv_ref, out_ref):
    g = pl.program_id(0)
    ones_col = jnp.ones((_n, 1), jnp.float32)
    mm = (((1,), (0,)), ((), ()))
    dl = (((1,), (1,)), ((), ()))
    ri = lax.broadcasted_iota(jnp.int32, (_n, _n), 0)
    ci = lax.broadcasted_iota(jnp.int32, (_n, _n), 1)
    lower = ri > ci

    cts = [(cnt0_ref[pl.ds(b * _n, _n), :]
            + cnt1_ref[pl.ds(b * _n, _n), :])[:, 0:_n] for b in range(_GB)]
    ygs = [y_ref[pl.ds(b * _n, _n), :] for b in range(_GB)]
    aggs = [lax.dot_general(cts[b], ygs[b], mm,
                            preferred_element_type=jnp.float32)
            for b in range(_GB)]
    degs = [lax.dot_general(cts[b], ones_col, mm,
                            preferred_element_type=jnp.float32) + 1.0
            for b in range(_GB)]
    hs = [jnp.maximum(
        (ygs[b] + aggs[b]) / degs[b] + b1_ref[...], 0.0)
        for b in range(_GB)]
    mus = [jnp.dot(hs[b], wmu_ref[...], preferred_element_type=jnp.float32)
           + bmu_ref[...] for b in range(_GB)]
    lvs = [jnp.dot(hs[b], wlv_ref[...], preferred_element_type=jnp.float32)
           + blv_ref[...] for b in range(_GB)]
    sigs = [jnp.exp(0.5 * lvs[b]) for b in range(_GB)]
    zs = [mus[b] + sigs[b] * eps_ref[pl.ds(b * _n, _n), :]
          for b in range(_GB)]
    Ls = [lax.dot_general(zs[b], zs[b], dl,
                          preferred_element_type=jnp.float32)
          for b in range(_GB)]

    kl_acc = jnp.zeros((_n, _DL), jnp.float32)
    val_acc = jnp.zeros((_n, _n), jnp.float32)
    for b in range(_GB):
        kl_acc += mus[b] * mus[b] + sigs[b] * sigs[b] - lvs[b]
        L = Ls[b]
        sp = jnp.maximum(L, 0.0) + jnp.log1p(jnp.exp(-jnp.abs(L)))
        val_acc += jnp.where(cts[b] > 0.0, L, 0.0) - sp

    klsum = 0.5 * (jnp.sum(kl_acc) - _GB * _n * _DL)
    logp = jnp.sum(jnp.where(lower, val_acc, 0.0))
    total = klsum - logp

    @pl.when(g == 0)
    def _():
        out_ref[0, 0] = 0.0

    out_ref[0, 0] += total * (1.0 / _G)


@jax.jit
def _tc_elbo(cnt0, cnt1, y, eps, b1, Wmu, bmu, Wlv, blv):
    blk = _GB * _n
    return pl.pallas_call(
        _tc_elbo_body,
        grid=(_G // _GB,),
        in_specs=[
            pl.BlockSpec((blk, _W), lambda g: (g, 0)),
            pl.BlockSpec((blk, _W), lambda g: (g, 0)),
            pl.BlockSpec((blk, _DH), lambda g: (g, 0)),
            pl.BlockSpec((blk, _DL), lambda g: (g, 0)),
            pl.BlockSpec((1, _DH), lambda g: (0, 0)),
            pl.BlockSpec((_DH, _DL), lambda g: (0, 0)),
            pl.BlockSpec((1, _DL), lambda g: (0, 0)),
            pl.BlockSpec((_DH, _DL), lambda g: (0, 0)),
            pl.BlockSpec((1, _DL), lambda g: (0, 0)),
        ],
        out_specs=pl.BlockSpec(memory_space=pltpu.SMEM),
        out_shape=jax.ShapeDtypeStruct((1, 1), jnp.float32),
    )(cnt0, cnt1, y, eps, b1, Wmu, bmu, Wlv, blv)


def kernel(x, edge_index, batch, eps, W1, b1, Wmu, bmu, Wlv, blv, pairs):
    c0, c1 = _sc_counts(edge_index.reshape(2 * _E))
    y = _xw1(x, W1)
    out = _tc_elbo(c0.reshape(_N, _W), c1.reshape(_N, _W), y, eps,
                   b1.reshape(1, _DH), Wmu, bmu.reshape(1, _DL),
                   Wlv, blv.reshape(1, _DL))
    return out[0, 0]


# speedup vs baseline: 1.3209x; 1.3142x over previous
import functools

import jax
import jax.numpy as jnp
from jax import lax
from jax.experimental import pallas as pl
from jax.experimental.pallas import tpu as pltpu
from jax.experimental.pallas import tpu_sc as plsc

_N = 10000
_G = 100
_n = 100
_DF = 128
_DH = 128
_DL = 64
_E = 160000
_W = 128
_TBL = _N * _W
_TBL_PAD = _TBL + 256
_NW = 32
_EPW = _E // _NW
_CHUNKS = 313
_ROWS = 40
_ZCH = 32000
_NCH = _TBL // _ZCH
_KMAX = 3
_OCH = 16000
_OK = 5
_GB = 20


def _sc_counts_body(ei_hbm, out0_hbm, out1_hbm, src_v, dst_v,
                    idx_v, val_v, zbuf, table_sh, sem_s, sem_sc):
    cid = lax.axis_index("c")
    sid = lax.axis_index("s")
    wid = cid * 16 + sid
    base = wid * _EPW

    ones16 = jnp.ones((16,), jnp.float32)
    zeros16 = jnp.zeros((16,), jnp.float32)
    trash16 = jnp.full((16,), _TBL, jnp.int32)
    lane = lax.broadcasted_iota(jnp.int32, (16,), 0)

    cp_s = pltpu.async_copy(ei_hbm.at[pl.ds(base, _EPW)],
                            src_v.at[pl.ds(0, _EPW)], sem_s)
    cp_d = pltpu.async_copy(ei_hbm.at[pl.ds(_E + base, _EPW)],
                            dst_v.at[pl.ds(0, _EPW)], sem_s)

    with jax.named_scope("p1_zbuf"):
        def _zb(i, carry):
            for c in range(8):
                zbuf[pl.ds(i * 128 + c * 16, 16)] = zeros16
            return carry
        lax.fori_loop(0, _ZCH // 128, _zb, 0)

    with jax.named_scope("p2_zerotable"):
        for k in range(_KMAX):
            ch = sid + 16 * k

            @pl.when(ch < _NCH)
            def _():
                pltpu.sync_copy(zbuf, table_sh.at[pl.ds(ch * _ZCH, _ZCH)])

        @pl.when(sid == 0)
        def _():
            pltpu.sync_copy(zbuf.at[pl.ds(0, 256)],
                            table_sh.at[pl.ds(_TBL, 256)])

    for i in range(8):
        val_v[pl.ds(i * 16, 16)] = ones16

    with jax.named_scope("p3_edgewait"):
        cp_s.wait()
        cp_d.wait()

    def _mod_n(v):
        q = lax.shift_right_logical(v * 5243, 19)
        return v - q * _n

    def _ixrow(r, carry):
        base16 = r * 128
        for c in range(8):
            off = base16 + c * 16
            s16 = src_v[pl.ds(off, 16)]
            d16 = dst_v[pl.ds(off, 16)]
            idx_v[r, pl.ds(c * 16, 16)] = d16 * _W + _mod_n(s16)
        return carry
    with jax.named_scope("p4_idx"):
        lax.fori_loop(0, _ROWS - 1, _ixrow, 0)
        s16 = src_v[pl.ds((_ROWS - 1) * 128, 16)]
        d16 = dst_v[pl.ds((_ROWS - 1) * 128, 16)]
        fl = d16 * _W + _mod_n(s16)
        fl = jnp.where(lane < _EPW - (_ROWS - 1) * 128, fl, _TBL)
        idx_v[_ROWS - 1, pl.ds(0, 16)] = fl
        for c in range(1, 8):
            idx_v[_ROWS - 1, pl.ds(c * 16, 16)] = trash16

    with jax.named_scope("p5_bar1"):
        plsc.subcore_barrier()

    with jax.named_scope("p6_scatter"):
        scps = [pltpu.async_copy(val_v, table_sh.at[idx_v.at[j]], sem_sc,
                                 add=True)
                for j in range(_ROWS)]
        for cp in scps:
            cp.wait()

    with jax.named_scope("p7_bar2"):
        plsc.subcore_barrier()

    with jax.named_scope("p8_copyout"):
        for k in range(_KMAX):
            ch = sid + 16 * k

            @pl.when(ch < _NCH)
            def _():
                pltpu.sync_copy(table_sh.at[pl.ds(ch * _ZCH, _ZCH)], zbuf)

                @pl.when(cid == 0)
                def _():
                    pltpu.sync_copy(zbuf,
                                    out0_hbm.at[pl.ds(ch * _ZCH, _ZCH)])

                @pl.when(cid == 1)
                def _():
                    pltpu.sync_copy(zbuf,
                                    out1_hbm.at[pl.ds(ch * _ZCH, _ZCH)])


@jax.jit
def _sc_counts(ei_flat):
    mesh = plsc.VectorSubcoreMesh(core_axis_name="c", subcore_axis_name="s")
    f = functools.partial(
        pl.kernel,
        mesh=mesh,
        out_type=(jax.ShapeDtypeStruct((_TBL,), jnp.float32),
                  jax.ShapeDtypeStruct((_TBL,), jnp.float32)),
        scratch_types=[
            pltpu.VMEM((_EPW + 16,), jnp.int32),
            pltpu.VMEM((_EPW + 16,), jnp.int32),
            pltpu.VMEM((_ROWS, 128), jnp.int32),
            pltpu.VMEM((128,), jnp.float32),
            pltpu.VMEM((_ZCH,), jnp.float32),
            pltpu.VMEM_SHARED((_TBL_PAD,), jnp.float32),
            pltpu.SemaphoreType.DMA,
            pltpu.SemaphoreType.DMA,
        ],
    )(_sc_counts_body)
    return f(ei_flat)


def _xw1_body(x_ref, w1_ref, y_ref):
    y_ref[...] = jnp.dot(x_ref[...], w1_ref[...],
                         preferred_element_type=jnp.float32)


@jax.jit
def _xw1(x, W1):
    blk = _GB * _n
    return pl.pallas_call(
        _xw1_body,
        grid=(_G // _GB,),
        in_specs=[
            pl.BlockSpec((blk, _DF), lambda g: (g, 0)),
            pl.BlockSpec((_DF, _DH), lambda g: (0, 0)),
        ],
        out_specs=pl.BlockSpec((blk, _DH), lambda g: (g, 0)),
        out_shape=jax.ShapeDtypeStruct((_N, _DH), jnp.float32),
    )(x, W1)


def _tc_elbo_body(cnt0_ref, cnt1_ref, y_ref, eps_ref, b1_ref,
                  wmu_ref, bmu_ref, wlv_ref, blv_ref, out_ref):
    g = pl.program_id(0)
    ones_col = jnp.ones((_n, 1), jnp.float32)
    mm = (((1,), (0,)), ((), ()))
    dl = (((1,), (1,)), ((), ()))
    ri = lax.broadcasted_iota(jnp.int32, (_n, _n), 0)
    ci = lax.broadcasted_iota(jnp.int32, (_n, _n), 1)
    lower = ri > ci

    cts = [(cnt0_ref[pl.ds(b * _n, _n), :]
            + cnt1_ref[pl.ds(b * _n, _n), :])[:, 0:_n] for b in range(_GB)]
    ygs = [y_ref[pl.ds(b * _n, _n), :] for b in range(_GB)]
    aggs = [lax.dot_general(cts[b], ygs[b], mm,
                            preferred_element_type=jnp.float32)
            for b in range(_GB)]
    degs = [lax.dot_general(cts[b], ones_col, mm,
                            preferred_element_type=jnp.float32) + 1.0
            for b in range(_GB)]
    hs = [jnp.maximum(
        (ygs[b] + aggs[b]) / degs[b] + b1_ref[...], 0.0)
        for b in range(_GB)]
    mus = [jnp.dot(hs[b], wmu_ref[...], preferred_element_type=jnp.float32)
           + bmu_ref[...] for b in range(_GB)]
    lvs = [jnp.dot(hs[b], wlv_ref[...], preferred_element_type=jnp.float32)
           + blv_ref[...] for b in range(_GB)]
    sigs = [jnp.exp(0.5 * lvs[b]) for b in range(_GB)]
    zs = [mus[b] + sigs[b] * eps_ref[pl.ds(b * _n, _n), :]
          for b in range(_GB)]
    Ls = [lax.dot_general(zs[b], zs[b], dl,
                          preferred_element_type=jnp.float32)
          for b in range(_GB)]

    kl_acc = jnp.zeros((_n, _DL), jnp.float32)
    val_acc = jnp.zeros((_n, _n), jnp.float32)
    for b in range(_GB):
        kl_acc += mus[b] * mus[b] + sigs[b] * sigs[b] - lvs[b]
        L = Ls[b]
        sp = jnp.maximum(L, 0.0) + jnp.log1p(jnp.exp(-jnp.abs(L)))
        val_acc += jnp.where(cts[b] > 0.0, L, 0.0) - sp

    klsum = 0.5 * (jnp.sum(kl_acc) - _GB * _n * _DL)
    logp = jnp.sum(jnp.where(lower, val_acc, 0.0))
    total = klsum - logp

    @pl.when(g == 0)
    def _():
        out_ref[0, 0] = 0.0

    out_ref[0, 0] += total * (1.0 / _G)


@jax.jit
def _tc_elbo(cnt0, cnt1, y, eps, b1, Wmu, bmu, Wlv, blv):
    blk = _GB * _n
    return pl.pallas_call(
        _tc_elbo_body,
        grid=(_G // _GB,),
        in_specs=[
            pl.BlockSpec((blk, _W), lambda g: (g, 0)),
            pl.BlockSpec((blk, _W), lambda g: (g, 0)),
            pl.BlockSpec((blk, _DH), lambda g: (g, 0)),
            pl.BlockSpec((blk, _DL), lambda g: (g, 0)),
            pl.BlockSpec((1, _DH), lambda g: (0, 0)),
            pl.BlockSpec((_DH, _DL), lambda g: (0, 0)),
            pl.BlockSpec((1, _DL), lambda g: (0, 0)),
            pl.BlockSpec((_DH, _DL), lambda g: (0, 0)),
            pl.BlockSpec((1, _DL), lambda g: (0, 0)),
        ],
        out_specs=pl.BlockSpec(memory_space=pltpu.SMEM),
        out_shape=jax.ShapeDtypeStruct((1, 1), jnp.float32),
    )(cnt0, cnt1, y, eps, b1, Wmu, bmu, Wlv, blv)


def kernel(x, edge_index, batch, eps, W1, b1, Wmu, bmu, Wlv, blv, pairs):
    c0, c1 = _sc_counts(edge_index.reshape(2 * _E))
    y = _xw1(x, W1)
    out = _tc_elbo(c0.reshape(_N, _W), c1.reshape(_N, _W), y, eps,
                   b1.reshape(1, _DH), Wmu, bmu.reshape(1, _DL),
                   Wlv, blv.reshape(1, _DL))
    return out[0, 0]
